# baseline (speedup 1.0000x reference)
import functools

import jax
import jax.numpy as jnp
from jax import lax
from jax.experimental import pallas as pl
from jax.experimental.pallas import tpu as pltpu
from jax.experimental.pallas import tpu_sc as plsc

N_NODES = 10000
N_EDGES = 320000
N_NODE_IN = 128
N_EDGE_IN = 16
N_NODE_OUT = 3
HID = 128

NC = 2
NS = 16
NW = NC * NS
PER_W = N_EDGES // NW
CHUNK = 80
N_CHUNKS = PER_W // CHUNK

NODES_PER_W = 312
LAST_EXTRA = N_NODES - NW * NODES_PER_W
CHUNKS_W = 142
PER_W_PAD = CHUNKS_W * CHUNK
ACC_ROWS = 10112
TRASH = 10016
ZROWS = ACC_ROWS // NS


@functools.cache
def _sc_mesh():
    return plsc.VectorSubcoreMesh(core_axis_name="c", subcore_axis_name="s",
                                  num_cores=NC, num_subcores=NS)


@functools.cache
def _sc_gather_kernel():
    @functools.partial(
        pl.kernel,
        out_type=(jax.ShapeDtypeStruct((N_EDGES, HID), jnp.float32),
                  jax.ShapeDtypeStruct((N_EDGES, HID), jnp.float32)),
        mesh=_sc_mesh(),
        scratch_types=[
            pltpu.VMEM((CHUNK,), jnp.int32),
            pltpu.VMEM((CHUNK,), jnp.int32),
            pltpu.VMEM((CHUNK, HID), jnp.float32),
            pltpu.VMEM((CHUNK, HID), jnp.float32),
            pltpu.SemaphoreType.DMA,
            pltpu.SemaphoreType.DMA,
        ],
    )
    def _sc_gather(x_hbm, src_hbm, dst_hbm, gs_hbm, gd_hbm,
                   idxs_v, idxd_v, rows_s, rows_d, sem_s, sem_d):
        cid = lax.axis_index("c")
        sid = lax.axis_index("s")
        base0 = (cid * NS + sid) * PER_W

        def body(i, carry):
            base = base0 + i * CHUNK
            pltpu.sync_copy(src_hbm.at[pl.ds(base, CHUNK)], idxs_v)
            pltpu.sync_copy(dst_hbm.at[pl.ds(base, CHUNK)], idxd_v)
            pltpu.async_copy(x_hbm.at[idxs_v], rows_s, sem_s).wait()
            pltpu.async_copy(x_hbm.at[idxd_v], rows_d, sem_d).wait()
            pltpu.sync_copy(rows_s, gs_hbm.at[pl.ds(base, CHUNK)])
            pltpu.sync_copy(rows_d, gd_hbm.at[pl.ds(base, CHUNK)])
            return carry

        lax.fori_loop(0, N_CHUNKS, body, 0)

    return _sc_gather


@functools.cache
def _sc_scatter_kernel():
    @functools.partial(
        pl.kernel,
        out_type=jax.ShapeDtypeStruct((N_NODES, HID), jnp.float32),
        mesh=_sc_mesh(),
        scratch_types=[
            pltpu.VMEM_SHARED((ACC_ROWS, HID), jnp.float32),
            pltpu.VMEM((CHUNK,), jnp.int32),
            pltpu.VMEM((CHUNK,), jnp.int32),
            pltpu.VMEM((CHUNK, HID), jnp.float32),
            pltpu.SemaphoreType.DMA,
        ],
    )
    def _sc_scatter(en_hbm, pperm_hbm, pdst_hbm, zeros_hbm, out_hbm,
                    agg_sh, idxp_v, idxd_v, rows_v, sem):
        cid = lax.axis_index("c")
        sid = lax.axis_index("s")
        w = cid * NS + sid
        zr = sid * ZROWS
        pltpu.sync_copy(zeros_hbm.at[pl.ds(zr, ZROWS)],
                        agg_sh.at[pl.ds(zr, ZROWS)])
        plsc.subcore_barrier()

        base0 = w * PER_W_PAD

        def body(i, carry):
            base = base0 + i * CHUNK
            pltpu.sync_copy(pperm_hbm.at[pl.ds(base, CHUNK)], idxp_v)
            pltpu.sync_copy(pdst_hbm.at[pl.ds(base, CHUNK)], idxd_v)
            pltpu.async_copy(en_hbm.at[idxp_v], rows_v, sem).wait()
            pltpu.sync_copy(rows_v, agg_sh.at[idxd_v], add=True)
            return carry

        lax.fori_loop(0, CHUNKS_W, body, 0)
        plsc.subcore_barrier()
        r0 = w * NODES_PER_W
        pltpu.sync_copy(agg_sh.at[pl.ds(r0, NODES_PER_W)],
                        out_hbm.at[pl.ds(r0, NODES_PER_W)])

        @pl.when(w == NW - 1)
        def _():
            pltpu.sync_copy(agg_sh.at[pl.ds(NW * NODES_PER_W, LAST_EXTRA)],
                            out_hbm.at[pl.ds(NW * NODES_PER_W, LAST_EXTRA)])

    return _sc_scatter


def _split3(a):
    a0 = a.astype(jnp.bfloat16)
    r = a - a0.astype(jnp.float32)
    a1 = r.astype(jnp.bfloat16)
    a2 = (r - a1.astype(jnp.float32)).astype(jnp.bfloat16)
    return a0, a1, a2


def _dot(a, w):
    return jnp.dot(a.astype(jnp.bfloat16), w.astype(jnp.bfloat16),
                   preferred_element_type=jnp.float32)


def _enc_node_body(x_ref, w1, b1, w2, b2, w3, b3, x_out):
    h = jnp.maximum(_dot(x_ref[...], w1[...]) + b1[...], 0.0)
    h = jnp.maximum(_dot(h, w2[...]) + b2[...], 0.0)
    x_out[...] = _dot(h, w3[...]) + b3[...]


def _enc_edge_body(e_ref, w1, b1, w2, b2, w3, b3, e_out):
    h = jnp.maximum(_dot(e_ref[...], w1[...]) + b1[...], 0.0)
    h = jnp.maximum(_dot(h, w2[...]) + b2[...], 0.0)
    e_out[...] = _dot(h, w3[...]) + b3[...]


def _edge_step_body(e_ref, gs_ref, gd_ref, w1, b1, w2, b2, w3, b3,
                    en_out, e_out):
    cat = jnp.concatenate([e_ref[...], gs_ref[...], gd_ref[...]], axis=1)
    h = jnp.maximum(_dot(cat, w1[...]) + b1[...], 0.0)
    h = jnp.maximum(_dot(h, w2[...]) + b2[...], 0.0)
    en = _dot(h, w3[...]) + b3[...]
    en_out[...] = en
    e_out[...] = e_ref[...] + en


def _node_step_body(x_ref, agg_ref, v1, c1, w2, b2, w3, b3, x_out):
    cat = jnp.concatenate([x_ref[...], agg_ref[...]], axis=1)
    h = jnp.maximum(_dot(cat, v1[...]) + c1[...], 0.0)
    h = jnp.maximum(_dot(h, w2[...]) + b2[...], 0.0)
    x_out[...] = x_ref[...] + (_dot(h, w3[...]) + b3[...])


def _dec_body(x_ref, w1, b1, w2, b2, w3, b3, y_out):
    h = jnp.maximum(_dot(x_ref[...], w1[...]) + b1[...], 0.0)
    h = jnp.maximum(_dot(h, w2[...]) + b2[...], 0.0)
    y_out[...] = _dot(h, w3[...]) + b3[...]


_E_BLK = 2560


def _enc_node_call(x, pr):
    return pl.pallas_call(
        _enc_node_body,
        out_shape=jax.ShapeDtypeStruct((N_NODES, HID), jnp.float32),
    )(x, *pr)


def _enc_edge_call(e, pr):
    grid = (N_EDGES // _E_BLK,)
    in_specs = [pl.BlockSpec((_E_BLK, N_EDGE_IN), lambda i: (i, 0))]
    for a in pr:
        in_specs.append(pl.BlockSpec(a.shape, lambda i: (0, 0)))
    return pl.pallas_call(
        _enc_edge_body,
        grid=grid,
        in_specs=in_specs,
        out_specs=pl.BlockSpec((_E_BLK, HID), lambda i: (i, 0)),
        out_shape=jax.ShapeDtypeStruct((N_EDGES, HID), jnp.float32),
    )(e, *pr)


def _edge_step_call(e, gs, gd, pr):
    grid = (N_EDGES // _E_BLK,)
    blk = pl.BlockSpec((_E_BLK, HID), lambda i: (i, 0))
    in_specs = [blk, blk, blk]
    for a in pr:
        in_specs.append(pl.BlockSpec(a.shape, lambda i: (0, 0)))
    out = (jax.ShapeDtypeStruct((N_EDGES, HID), jnp.float32),) * 2
    return pl.pallas_call(
        _edge_step_body,
        grid=grid,
        in_specs=in_specs,
        out_specs=(blk, blk),
        out_shape=out,
    )(e, gs, gd, *pr)


def _node_step_call(x, agg, pr):
    return pl.pallas_call(
        _node_step_body,
        out_shape=jax.ShapeDtypeStruct((N_NODES, HID), jnp.float32),
    )(x, agg, *pr)


def _dec_call(x, pr):
    return pl.pallas_call(
        _dec_body,
        out_shape=jax.ShapeDtypeStruct((N_NODES, N_NODE_OUT), jnp.float32),
    )(x, *pr)


def _flat(params_list):
    out = []
    for w, b in params_list:
        out.append(w)
        out.append(b.reshape(1, -1))
    return out


def kernel(node_features_in, edge_features_in, edges_indexes, params):
    src = edges_indexes[:, 0]
    dst = edges_indexes[:, 1]

    order = jnp.argsort(dst, stable=True).astype(jnp.int32)
    sdst = dst[order]
    cuts = jnp.concatenate(
        [jnp.arange(NW, dtype=jnp.int32) * NODES_PER_W,
         jnp.array([N_NODES], jnp.int32)])
    bounds = jnp.searchsorted(sdst, cuts, side='left').astype(jnp.int32)
    pos = bounds[:-1, None] + jnp.arange(PER_W_PAD, dtype=jnp.int32)[None, :]
    valid = pos < bounds[1:, None]
    posc = jnp.minimum(pos, N_EDGES - 1)
    pperm = jnp.where(valid, order[posc], 0).reshape(-1)
    pdst = jnp.where(valid, sdst[posc], TRASH).reshape(-1)
    zeros_acc = jnp.zeros((ACC_ROWS, HID), jnp.float32)

    gins = params['gins']
    edge_pr = [_flat(g['edge']) for g in gins]
    node_pr = [_flat(g['node']) for g in gins]

    x = _enc_node_call(node_features_in, _flat(params['enc_node']))
    e = _enc_edge_call(edge_features_in, _flat(params['enc_edge']))

    for s in range(len(gins)):
        gs, gd = _sc_gather_kernel()(x, src, dst)
        en, e = _edge_step_call(e, gs, gd, edge_pr[s])
        agg = _sc_scatter_kernel()(en, pperm, pdst, zeros_acc)
        x = _node_step_call(x, agg, node_pr[s])

    return _dec_call(x, _flat(params['dec']))
